# Initial kernel scaffold; baseline (speedup 1.0000x reference)
import jax
import jax.numpy as jnp
from jax.experimental import pallas as pl


def kernel(x, w1_low, w2_low, w1_high, w2_high, w_fuse, b_fuse, gamma, beta):
    raise NotImplementedError("write your pallas kernel here")


# speedup vs baseline: 17.5779x; 17.5779x over previous
import numpy as np
import jax
import jax.numpy as jnp
from jax.experimental import pallas as pl
from jax.experimental.pallas import tpu as pltpu

_B, _C, _D, _H, _W = 2, 32, 96, 96, 96
_S = 0.7071067811865476
_S3 = _S * _S * _S
_EPS = 1e-5
_DO = _D // 2
_NPOS = (_D // 2) * (_H // 2) * (_W // 2)
_D_BLK1 = 8


def _sums_kernel(x_ref, out_ref, acc_ref):
    j = pl.program_id(1)
    nj = pl.num_programs(1)

    @pl.when(j == 0)
    def _():
        acc_ref[...] = jnp.zeros_like(acc_ref)

    xb = x_ref[0]
    q = xb[:, 0] + xb[:, 1]
    r = xb[:, 0] - xb[:, 1]
    for i in range(1, _D_BLK1 // 2):
        q = q + xb[:, 2 * i] + xb[:, 2 * i + 1]
        r = r + xb[:, 2 * i] - xb[:, 2 * i + 1]

    hi = jax.lax.broadcasted_iota(jnp.int32, (1, _H, _W), 1)
    sh = jnp.where(hi % 2 == 0, 1.0, -1.0)
    wi = jax.lax.broadcasted_iota(jnp.int32, (1, _W), 1)
    sw = jnp.where(wi % 2 == 0, 1.0, -1.0)

    cq = q.sum(axis=1)
    cqh = (q * sh).sum(axis=1)
    cr = r.sum(axis=1)
    crh = (r * sh).sum(axis=1)

    cols = (cq, cqh, cr, crh)
    for i, col in enumerate(cols):
        sa = col.sum(axis=1)
        sd = (col * sw).sum(axis=1)
        acc_ref[2 * i:2 * i + 1, :] += sa[None, :]
        acc_ref[2 * i + 1:2 * i + 2, :] += sd[None, :]

    @pl.when(j == nj - 1)
    def _():
        for k in range(8):
            out_ref[:, :, 32 * k:32 * (k + 1)] = acc_ref[k:k + 1, :][None]


def _main_kernel(x_ref, sums_ref, pw_ref, w1l_ref, w2l_ref, w1h_ref,
                 w2h_ref, wf_ref, z_ref, st_ref, a_scr, st_scr):
    j = pl.program_id(1)
    nj = pl.num_programs(1)

    @pl.when(j == 0)
    def _():
        st_scr[...] = jnp.zeros_like(st_scr)
        means = sums_ref[0] * (_S3 / float(_NPOS))
        y_low = means[:, 0:32]
        y_high = means[:, 32:256]
        h1 = jnp.maximum(
            jnp.dot(y_low, w1l_ref[...], preferred_element_type=jnp.float32), 0.0)
        s_low = jax.nn.sigmoid(
            jnp.dot(h1, w2l_ref[...], preferred_element_type=jnp.float32))
        h2 = jnp.maximum(
            jnp.dot(y_high, w1h_ref[...], preferred_element_type=jnp.float32), 0.0)
        s_high = jax.nn.sigmoid(
            jnp.dot(h2, w2h_ref[...], preferred_element_type=jnp.float32))
        s_full = jnp.concatenate([s_low, s_high], axis=1)
        a_scr[...] = (wf_ref[...] * s_full * _S3).astype(jnp.bfloat16)

    x0 = x_ref[0, :, 0]
    x1 = x_ref[0, :, 1]
    pw = pw_ref[...]
    bands = []
    for y in (x0 + x1, x0 - x1):
        d1 = jax.lax.dot_general(y, pw, (((1,), (0,)), ((), ())),
                                 preferred_element_type=jnp.float32)
        d2 = jax.lax.dot_general(d1, pw, (((1,), (0,)), ((), ())),
                                 preferred_element_type=jnp.float32)
        bands.append(d2[:, 0:48, 0:48].astype(jnp.bfloat16))
        bands.append(d2[:, 0:48, 48:96].astype(jnp.bfloat16))
        bands.append(d2[:, 48:96, 0:48].astype(jnp.bfloat16))
        bands.append(d2[:, 48:96, 48:96].astype(jnp.bfloat16))
    u = jnp.concatenate(bands, axis=0)

    z = jax.lax.dot_general(a_scr[...], u, (((1,), (0,)), ((), ())),
                            preferred_element_type=jnp.float32)
    z_ref[0, :, 0] = z
    st_scr[0:1, :] += z.sum(axis=(1, 2))[None]
    st_scr[1:2, :] += (z * z).sum(axis=(1, 2))[None]

    @pl.when(j == nj - 1)
    def _():
        st_ref[0] = st_scr[...]


def _bn_kernel(z_ref, g_ref, b_ref, o_ref):
    zb = z_ref[0]
    g = g_ref[...][:, None, None, :]
    b = b_ref[...][:, None, None, :]
    o_ref[0] = jnp.maximum(zb * g + b, 0.0)


def _haar_butterfly():
    p = np.zeros((96, 96), np.float32)
    i = np.arange(48)
    p[2 * i, i] = 1.0
    p[2 * i + 1, i] = 1.0
    p[2 * i, 48 + i] = 1.0
    p[2 * i + 1, 48 + i] = -1.0
    return p


def kernel(x, w1_low, w2_low, w1_high, w2_high, w_fuse, b_fuse, gamma, beta):
    del b_fuse

    c_ = np.arange(32)
    perm = np.empty(256, np.int32)
    perm[0:32] = c_
    for k in range(1, 8):
        perm[32 * k + c_] = 32 + 7 * c_ + (k - 1)
    perm_h = perm[32:] - 32

    pw = jnp.asarray(_haar_butterfly())
    wf_p = jnp.asarray(w_fuse)[:, perm]
    w1l_t = w1_low.T
    w2l_t = w2_low.T
    w1h_t = w1_high[:, perm_h].T
    w2h_t = w2_high[perm_h, :].T

    sums = pl.pallas_call(
        _sums_kernel,
        out_shape=jax.ShapeDtypeStruct((_B, 1, 256), jnp.float32),
        grid=(_B, _D // _D_BLK1),
        in_specs=[pl.BlockSpec((1, _C, _D_BLK1, _H, _W),
                               lambda b, j: (b, 0, j, 0, 0))],
        out_specs=pl.BlockSpec((1, 1, 256), lambda b, j: (b, 0, 0)),
        scratch_shapes=[pltpu.VMEM((8, 32), jnp.float32)],
        compiler_params=pltpu.CompilerParams(
            dimension_semantics=("parallel", "arbitrary")),
        name="haar_band_sums",
    )(x)

    z, st = pl.pallas_call(
        _main_kernel,
        out_shape=(jax.ShapeDtypeStruct((_B, _C, _DO, 48, 48), jnp.float32),
                   jax.ShapeDtypeStruct((_B, 8, 32), jnp.float32)),
        grid=(_B, _DO),
        in_specs=[
            pl.BlockSpec((1, _C, 2, _H, _W), lambda b, j: (b, 0, j, 0, 0)),
            pl.BlockSpec((1, 1, 256), lambda b, j: (b, 0, 0)),
            pl.BlockSpec((96, 96), lambda b, j: (0, 0)),
            pl.BlockSpec((32, 2), lambda b, j: (0, 0)),
            pl.BlockSpec((2, 32), lambda b, j: (0, 0)),
            pl.BlockSpec((224, 14), lambda b, j: (0, 0)),
            pl.BlockSpec((14, 224), lambda b, j: (0, 0)),
            pl.BlockSpec((32, 256), lambda b, j: (0, 0)),
        ],
        out_specs=(pl.BlockSpec((1, _C, 1, 48, 48), lambda b, j: (b, 0, j, 0, 0)),
                   pl.BlockSpec((1, 8, 32), lambda b, j: (b, 0, 0))),
        scratch_shapes=[pltpu.VMEM((32, 256), jnp.bfloat16),
                        pltpu.VMEM((8, 32), jnp.float32)],
        compiler_params=pltpu.CompilerParams(
            dimension_semantics=("parallel", "arbitrary")),
        name="haar_attn_fuse",
    )(x, sums, pw, w1l_t, w2l_t, w1h_t, w2h_t, wf_p)

    cnt = float(_B * _NPOS)
    ssum = st[:, 0, :].sum(axis=0)
    ssq = st[:, 1, :].sum(axis=0)
    mu = ssum / cnt
    var = ssq / cnt - mu * mu
    inv = jax.lax.rsqrt(var + _EPS)
    ga = gamma * inv
    bb = beta - mu * ga
    ga48 = jnp.broadcast_to(ga[:, None], (32, 48))
    bb48 = jnp.broadcast_to(bb[:, None], (32, 48))

    d_blk3 = 12
    out = pl.pallas_call(
        _bn_kernel,
        out_shape=jax.ShapeDtypeStruct((_B, _C, _DO, 48, 48), jnp.float32),
        grid=(_B, _DO // d_blk3),
        in_specs=[
            pl.BlockSpec((1, _C, d_blk3, 48, 48), lambda b, j: (b, 0, j, 0, 0)),
            pl.BlockSpec((32, 48), lambda b, j: (0, 0)),
            pl.BlockSpec((32, 48), lambda b, j: (0, 0)),
        ],
        out_specs=pl.BlockSpec((1, _C, d_blk3, 48, 48),
                               lambda b, j: (b, 0, j, 0, 0)),
        compiler_params=pltpu.CompilerParams(
            dimension_semantics=("parallel", "arbitrary")),
        name="bn_relu",
    )(z, ga48, bb48)
    return out


# speedup vs baseline: 18.0633x; 1.0276x over previous
import numpy as np
import jax
import jax.numpy as jnp
from jax.experimental import pallas as pl
from jax.experimental.pallas import tpu as pltpu

_B, _C, _D, _H, _W = 2, 32, 96, 96, 96
_S = 0.7071067811865476
_S3 = _S * _S * _S
_EPS = 1e-5
_DO = _D // 2
_NPOS = (_D // 2) * (_H // 2) * (_W // 2)
_D_BLK1 = 8


def _sums_kernel(x_ref, out_ref, acc_ref):
    j = pl.program_id(1)
    nj = pl.num_programs(1)

    @pl.when(j == 0)
    def _():
        acc_ref[...] = jnp.zeros_like(acc_ref)

    xb = x_ref[0]
    q = xb[:, 0] + xb[:, 1]
    r = xb[:, 0] - xb[:, 1]
    for i in range(1, _D_BLK1 // 2):
        q = q + xb[:, 2 * i] + xb[:, 2 * i + 1]
        r = r + xb[:, 2 * i] - xb[:, 2 * i + 1]

    hi = jax.lax.broadcasted_iota(jnp.int32, (1, _H, _W), 1)
    sh = jnp.where(hi % 2 == 0, 1.0, -1.0)
    wi = jax.lax.broadcasted_iota(jnp.int32, (1, _W), 1)
    sw = jnp.where(wi % 2 == 0, 1.0, -1.0)

    cq = q.sum(axis=1)
    cqh = (q * sh).sum(axis=1)
    cr = r.sum(axis=1)
    crh = (r * sh).sum(axis=1)

    cols = (cq, cqh, cr, crh)
    for i, col in enumerate(cols):
        sa = col.sum(axis=1)
        sd = (col * sw).sum(axis=1)
        acc_ref[2 * i:2 * i + 1, :] += sa[None, :]
        acc_ref[2 * i + 1:2 * i + 2, :] += sd[None, :]

    @pl.when(j == nj - 1)
    def _():
        for k in range(8):
            out_ref[:, :, 32 * k:32 * (k + 1)] = acc_ref[k:k + 1, :][None]


def _main_kernel(x_ref, sums_ref, pw_ref, w1l_ref, w2l_ref, w1h_ref,
                 w2h_ref, wf_ref, z_ref, st_ref, sf_ref, a_scr, st_scr, sf_scr):
    j = pl.program_id(1)
    nj = pl.num_programs(1)

    @pl.when(j == 0)
    def _():
        st_scr[...] = jnp.zeros_like(st_scr)
        means = sums_ref[0] * (_S3 / float(_NPOS))
        y_low = means[:, 0:32]
        y_high = means[:, 32:256]
        h1 = jnp.maximum(
            jnp.dot(y_low, w1l_ref[...], preferred_element_type=jnp.float32), 0.0)
        s_low = jax.nn.sigmoid(
            jnp.dot(h1, w2l_ref[...], preferred_element_type=jnp.float32))
        h2 = jnp.maximum(
            jnp.dot(y_high, w1h_ref[...], preferred_element_type=jnp.float32), 0.0)
        s_high = jax.nn.sigmoid(
            jnp.dot(h2, w2h_ref[...], preferred_element_type=jnp.float32))
        s_full = jnp.concatenate([s_low, s_high], axis=1)
        sf_scr[...] = s_full
        a_scr[...] = (wf_ref[...] * s_full * _S3).astype(jnp.bfloat16)

    x0 = x_ref[0, :, 0]
    x1 = x_ref[0, :, 1]
    pw = pw_ref[...]
    bands = []
    for y in (x0 + x1, x0 - x1):
        d1 = jax.lax.dot_general(y, pw, (((1,), (0,)), ((), ())),
                                 preferred_element_type=jnp.float32)
        d2 = jax.lax.dot_general(d1, pw, (((1,), (0,)), ((), ())),
                                 preferred_element_type=jnp.float32)
        bands.append(d2[:, 0:48, 0:48].astype(jnp.bfloat16))
        bands.append(d2[:, 0:48, 48:96].astype(jnp.bfloat16))
        bands.append(d2[:, 48:96, 0:48].astype(jnp.bfloat16))
        bands.append(d2[:, 48:96, 48:96].astype(jnp.bfloat16))
    u = jnp.concatenate(bands, axis=0)

    z = jax.lax.dot_general(a_scr[...], u, (((1,), (0,)), ((), ())),
                            preferred_element_type=jnp.float32)
    z_ref[0, :, 0] = z.astype(jnp.bfloat16)
    st_scr[0:1, :] += (z * z).sum(axis=(1, 2))[None]

    @pl.when(j == nj - 1)
    def _():
        st_ref[0] = st_scr[...]
        sf_ref[0] = sf_scr[...]


def _bn_kernel(z_ref, g_ref, b_ref, o_ref):
    zb = z_ref[0].astype(jnp.float32)
    g = g_ref[...][:, None, None, :]
    b = b_ref[...][:, None, None, :]
    o_ref[0] = jnp.maximum(zb * g + b, 0.0)


def _haar_butterfly():
    p = np.zeros((96, 96), np.float32)
    i = np.arange(48)
    p[2 * i, i] = 1.0
    p[2 * i + 1, i] = 1.0
    p[2 * i, 48 + i] = 1.0
    p[2 * i + 1, 48 + i] = -1.0
    return p


def kernel(x, w1_low, w2_low, w1_high, w2_high, w_fuse, b_fuse, gamma, beta):
    del b_fuse

    c_ = np.arange(32)
    perm = np.empty(256, np.int32)
    perm[0:32] = c_
    for k in range(1, 8):
        perm[32 * k + c_] = 32 + 7 * c_ + (k - 1)
    perm_h = perm[32:] - 32

    pw = jnp.asarray(_haar_butterfly())
    wf_p = jnp.asarray(w_fuse)[:, perm]
    w1l_t = w1_low.T
    w2l_t = w2_low.T
    w1h_t = w1_high[:, perm_h].T
    w2h_t = w2_high[perm_h, :].T

    sums = pl.pallas_call(
        _sums_kernel,
        out_shape=jax.ShapeDtypeStruct((_B, 1, 256), jnp.float32),
        grid=(_B, _D // _D_BLK1),
        in_specs=[pl.BlockSpec((1, _C, _D_BLK1, _H, _W),
                               lambda b, j: (b, 0, j, 0, 0))],
        out_specs=pl.BlockSpec((1, 1, 256), lambda b, j: (b, 0, 0)),
        scratch_shapes=[pltpu.VMEM((8, 32), jnp.float32)],
        compiler_params=pltpu.CompilerParams(
            dimension_semantics=("parallel", "arbitrary")),
        name="haar_band_sums",
    )(x)

    z, st, sf = pl.pallas_call(
        _main_kernel,
        out_shape=(jax.ShapeDtypeStruct((_B, _C, _DO, 48, 48), jnp.bfloat16),
                   jax.ShapeDtypeStruct((_B, 8, 32), jnp.float32),
                   jax.ShapeDtypeStruct((_B, 1, 256), jnp.float32)),
        grid=(_B, _DO),
        in_specs=[
            pl.BlockSpec((1, _C, 2, _H, _W), lambda b, j: (b, 0, j, 0, 0)),
            pl.BlockSpec((1, 1, 256), lambda b, j: (b, 0, 0)),
            pl.BlockSpec((96, 96), lambda b, j: (0, 0)),
            pl.BlockSpec((32, 2), lambda b, j: (0, 0)),
            pl.BlockSpec((2, 32), lambda b, j: (0, 0)),
            pl.BlockSpec((224, 14), lambda b, j: (0, 0)),
            pl.BlockSpec((14, 224), lambda b, j: (0, 0)),
            pl.BlockSpec((32, 256), lambda b, j: (0, 0)),
        ],
        out_specs=(pl.BlockSpec((1, _C, 1, 48, 48), lambda b, j: (b, 0, j, 0, 0)),
                   pl.BlockSpec((1, 8, 32), lambda b, j: (b, 0, 0)),
                   pl.BlockSpec((1, 1, 256), lambda b, j: (b, 0, 0))),
        scratch_shapes=[pltpu.VMEM((32, 256), jnp.bfloat16),
                        pltpu.VMEM((8, 32), jnp.float32),
                        pltpu.VMEM((1, 256), jnp.float32)],
        compiler_params=pltpu.CompilerParams(
            dimension_semantics=("parallel", "arbitrary")),
        name="haar_attn_fuse",
    )(x, sums, pw, w1l_t, w2l_t, w1h_t, w2h_t, wf_p)

    cnt = float(_B * _NPOS)
    a_full = wf_p[None, :, :] * sf[:, 0, None, :] * _S3
    ssum = jnp.einsum('boc,bc->o', a_full, sums[:, 0, :])
    ssq = st[:, 0, :].sum(axis=0)
    mu = ssum / cnt
    var = ssq / cnt - mu * mu
    inv = jax.lax.rsqrt(var + _EPS)
    ga = gamma * inv
    bb = beta - mu * ga
    ga48 = jnp.broadcast_to(ga[:, None], (32, 48))
    bb48 = jnp.broadcast_to(bb[:, None], (32, 48))

    d_blk3 = 12
    out = pl.pallas_call(
        _bn_kernel,
        out_shape=jax.ShapeDtypeStruct((_B, _C, _DO, 48, 48), jnp.float32),
        grid=(_B, _DO // d_blk3),
        in_specs=[
            pl.BlockSpec((1, _C, d_blk3, 48, 48), lambda b, j: (b, 0, j, 0, 0)),
            pl.BlockSpec((32, 48), lambda b, j: (0, 0)),
            pl.BlockSpec((32, 48), lambda b, j: (0, 0)),
        ],
        out_specs=pl.BlockSpec((1, _C, d_blk3, 48, 48),
                               lambda b, j: (b, 0, j, 0, 0)),
        compiler_params=pltpu.CompilerParams(
            dimension_semantics=("parallel", "arbitrary")),
        name="bn_relu",
    )(z, ga48, bb48)
    return out


# speedup vs baseline: 19.2855x; 1.0677x over previous
import numpy as np
import jax
import jax.numpy as jnp
from jax.experimental import pallas as pl
from jax.experimental.pallas import tpu as pltpu

_B, _C, _D, _H, _W = 2, 32, 96, 96, 96
_S = 0.7071067811865476
_S3 = _S * _S * _S
_EPS = 1e-5
_DO = _D // 2
_NPOS = (_D // 2) * (_H // 2) * (_W // 2)
_D_BLK1 = 8


def _sums_kernel(x_ref, out_ref, acc_ref):
    j = pl.program_id(1)
    nj = pl.num_programs(1)

    @pl.when(j == 0)
    def _():
        acc_ref[...] = jnp.zeros_like(acc_ref)

    xb = x_ref[0]
    q = xb[:, 0] + xb[:, 1]
    r = xb[:, 0] - xb[:, 1]
    for i in range(1, _D_BLK1 // 2):
        q = q + xb[:, 2 * i] + xb[:, 2 * i + 1]
        r = r + xb[:, 2 * i] - xb[:, 2 * i + 1]

    hi = jax.lax.broadcasted_iota(jnp.int32, (1, _H, _W), 1)
    sh = jnp.where(hi % 2 == 0, 1.0, -1.0)
    wi = jax.lax.broadcasted_iota(jnp.int32, (1, _W), 1)
    sw = jnp.where(wi % 2 == 0, 1.0, -1.0)

    cq = q.sum(axis=1)
    cqh = (q * sh).sum(axis=1)
    cr = r.sum(axis=1)
    crh = (r * sh).sum(axis=1)

    cols = (cq, cqh, cr, crh)
    for i, col in enumerate(cols):
        sa = col.sum(axis=1)
        sd = (col * sw).sum(axis=1)
        acc_ref[2 * i:2 * i + 1, :] += sa[None, :]
        acc_ref[2 * i + 1:2 * i + 2, :] += sd[None, :]

    @pl.when(j == nj - 1)
    def _():
        for k in range(8):
            out_ref[:, :, 32 * k:32 * (k + 1)] = acc_ref[k:k + 1, :][None]


def _main_kernel(x_ref, sums_ref, pw_ref, w1l_ref, w2l_ref, w1h_ref,
                 w2h_ref, wf_ref, z_ref, st_ref, sf_ref, a_scr, st_scr, sf_scr):
    j = pl.program_id(1)
    nj = pl.num_programs(1)

    @pl.when(j == 0)
    def _():
        st_scr[...] = jnp.zeros_like(st_scr)
        means = sums_ref[0] * (_S3 / float(_NPOS))
        y_low = means[:, 0:32]
        y_high = means[:, 32:256]
        h1 = jnp.maximum(
            jnp.dot(y_low, w1l_ref[...], preferred_element_type=jnp.float32), 0.0)
        s_low = jax.nn.sigmoid(
            jnp.dot(h1, w2l_ref[...], preferred_element_type=jnp.float32))
        h2 = jnp.maximum(
            jnp.dot(y_high, w1h_ref[...], preferred_element_type=jnp.float32), 0.0)
        s_high = jax.nn.sigmoid(
            jnp.dot(h2, w2h_ref[...], preferred_element_type=jnp.float32))
        s_full = jnp.concatenate([s_low, s_high], axis=1)
        sf_scr[...] = s_full
        a_scr[...] = (wf_ref[...] * s_full * _S3).astype(jnp.bfloat16)

    x0 = x_ref[0, :, 0].astype(jnp.bfloat16)
    x1 = x_ref[0, :, 1].astype(jnp.bfloat16)
    pw = pw_ref[...]
    bands = []
    for y in (x0 + x1, x0 - x1):
        d1 = jax.lax.dot_general(y, pw, (((1,), (0,)), ((), ())),
                                 preferred_element_type=jnp.float32)
        d2 = jax.lax.dot_general(d1.astype(jnp.bfloat16), pw,
                                 (((1,), (0,)), ((), ())),
                                 preferred_element_type=jnp.float32)
        bands.append(d2[:, 0:48, 0:48].astype(jnp.bfloat16))
        bands.append(d2[:, 0:48, 48:96].astype(jnp.bfloat16))
        bands.append(d2[:, 48:96, 0:48].astype(jnp.bfloat16))
        bands.append(d2[:, 48:96, 48:96].astype(jnp.bfloat16))
    u = jnp.concatenate(bands, axis=0)

    z = jax.lax.dot_general(a_scr[...], u, (((1,), (0,)), ((), ())),
                            preferred_element_type=jnp.float32)
    z_ref[0, :, 0] = z.astype(jnp.bfloat16)
    st_scr[0:1, :] += (z * z).sum(axis=(1, 2))[None]

    @pl.when(j == nj - 1)
    def _():
        st_ref[0] = st_scr[...]
        sf_ref[0] = sf_scr[...]


def _bn_kernel(z_ref, g_ref, b_ref, o_ref):
    zb = z_ref[0].astype(jnp.float32)
    g = g_ref[...][:, None, None, :]
    b = b_ref[...][:, None, None, :]
    o_ref[0] = jnp.maximum(zb * g + b, 0.0)


def _haar_butterfly():
    p = np.zeros((96, 96), np.float32)
    i = np.arange(48)
    p[2 * i, i] = 1.0
    p[2 * i + 1, i] = 1.0
    p[2 * i, 48 + i] = 1.0
    p[2 * i + 1, 48 + i] = -1.0
    return p


def kernel(x, w1_low, w2_low, w1_high, w2_high, w_fuse, b_fuse, gamma, beta):
    del b_fuse

    c_ = np.arange(32)
    perm = np.empty(256, np.int32)
    perm[0:32] = c_
    for k in range(1, 8):
        perm[32 * k + c_] = 32 + 7 * c_ + (k - 1)
    perm_h = perm[32:] - 32

    pw = jnp.asarray(_haar_butterfly()).astype(jnp.bfloat16)
    wf_p = jnp.asarray(w_fuse)[:, perm]
    w1l_t = w1_low.T
    w2l_t = w2_low.T
    w1h_t = w1_high[:, perm_h].T
    w2h_t = w2_high[perm_h, :].T

    sums = pl.pallas_call(
        _sums_kernel,
        out_shape=jax.ShapeDtypeStruct((_B, 1, 256), jnp.float32),
        grid=(_B, _D // _D_BLK1),
        in_specs=[pl.BlockSpec((1, _C, _D_BLK1, _H, _W),
                               lambda b, j: (b, 0, j, 0, 0))],
        out_specs=pl.BlockSpec((1, 1, 256), lambda b, j: (b, 0, 0)),
        scratch_shapes=[pltpu.VMEM((8, 32), jnp.float32)],
        compiler_params=pltpu.CompilerParams(
            dimension_semantics=("parallel", "arbitrary")),
        name="haar_band_sums",
    )(x)

    z, st, sf = pl.pallas_call(
        _main_kernel,
        out_shape=(jax.ShapeDtypeStruct((_B, _C, _DO, 48, 48), jnp.bfloat16),
                   jax.ShapeDtypeStruct((_B, 8, 32), jnp.float32),
                   jax.ShapeDtypeStruct((_B, 1, 256), jnp.float32)),
        grid=(_B, _DO),
        in_specs=[
            pl.BlockSpec((1, _C, 2, _H, _W), lambda b, j: (b, 0, j, 0, 0)),
            pl.BlockSpec((1, 1, 256), lambda b, j: (b, 0, 0)),
            pl.BlockSpec((96, 96), lambda b, j: (0, 0)),
            pl.BlockSpec((32, 2), lambda b, j: (0, 0)),
            pl.BlockSpec((2, 32), lambda b, j: (0, 0)),
            pl.BlockSpec((224, 14), lambda b, j: (0, 0)),
            pl.BlockSpec((14, 224), lambda b, j: (0, 0)),
            pl.BlockSpec((32, 256), lambda b, j: (0, 0)),
        ],
        out_specs=(pl.BlockSpec((1, _C, 1, 48, 48), lambda b, j: (b, 0, j, 0, 0)),
                   pl.BlockSpec((1, 8, 32), lambda b, j: (b, 0, 0)),
                   pl.BlockSpec((1, 1, 256), lambda b, j: (b, 0, 0))),
        scratch_shapes=[pltpu.VMEM((32, 256), jnp.bfloat16),
                        pltpu.VMEM((8, 32), jnp.float32),
                        pltpu.VMEM((1, 256), jnp.float32)],
        compiler_params=pltpu.CompilerParams(
            dimension_semantics=("parallel", "arbitrary")),
        name="haar_attn_fuse",
    )(x, sums, pw, w1l_t, w2l_t, w1h_t, w2h_t, wf_p)

    cnt = float(_B * _NPOS)
    a_full = wf_p[None, :, :] * sf[:, 0, None, :] * _S3
    ssum = jnp.einsum('boc,bc->o', a_full, sums[:, 0, :])
    ssq = st[:, 0, :].sum(axis=0)
    mu = ssum / cnt
    var = ssq / cnt - mu * mu
    inv = jax.lax.rsqrt(var + _EPS)
    ga = gamma * inv
    bb = beta - mu * ga
    ga48 = jnp.broadcast_to(ga[:, None], (32, 48))
    bb48 = jnp.broadcast_to(bb[:, None], (32, 48))

    d_blk3 = 12
    out = pl.pallas_call(
        _bn_kernel,
        out_shape=jax.ShapeDtypeStruct((_B, _C, _DO, 48, 48), jnp.float32),
        grid=(_B, _DO // d_blk3),
        in_specs=[
            pl.BlockSpec((1, _C, d_blk3, 48, 48), lambda b, j: (b, 0, j, 0, 0)),
            pl.BlockSpec((32, 48), lambda b, j: (0, 0)),
            pl.BlockSpec((32, 48), lambda b, j: (0, 0)),
        ],
        out_specs=pl.BlockSpec((1, _C, d_blk3, 48, 48),
                               lambda b, j: (b, 0, j, 0, 0)),
        compiler_params=pltpu.CompilerParams(
            dimension_semantics=("parallel", "arbitrary")),
        name="bn_relu",
    )(z, ga48, bb48)
    return out
